# baseline (speedup 1.0000x reference)
import jax
import jax.numpy as jnp
from jax import lax
from jax.experimental import pallas as pl
from jax.experimental.pallas import tpu as pltpu

_EPS = 1e-12
_NBUF = 3


def _body(grid_hbm, row_ref, col_ref, tok_ref, lnw_ref, lnb_ref, out_hbm,
          buf0, buf1, buf2, ost0, ost1, insem, outsem):
    bufs = (buf0, buf1, buf2)
    osts = (ost0, ost1)
    B = grid_hbm.shape[0]
    f = grid_hbm.shape[1]

    def in_copy(b):
        return pltpu.make_async_copy(
            grid_hbm.at[b], bufs[b % _NBUF], insem.at[b % _NBUF])

    def out_copy(b):
        return pltpu.make_async_copy(
            osts[b % 2], out_hbm.at[b], outsem.at[b % 2])

    row = row_ref[...]
    col = col_ref[...]
    tok = tok_ref[...]
    bias = row[:, None, :] + (col + tok)[None, :, :]
    lnw = lnw_ref[...][None, :, :]
    lnb = lnb_ref[...][None, :, :]

    in_copy(0).start()
    in_copy(1).start()
    for b in range(B):
        in_copy(b).wait()
        if b + 2 < B:
            in_copy(b + 2).start()
        if b >= 2:
            out_copy(b - 2).wait()
        x = bufs[b % _NBUF][...]
        m = jnp.sum(x, axis=0) * (1.0 / f)
        e = m + bias
        mu = jnp.mean(e, axis=-1, keepdims=True)
        d = e - mu
        var = jnp.mean(d * d, axis=-1, keepdims=True)
        inv = lax.rsqrt(var + _EPS)
        osts[b % 2][...] = d * inv * lnw + lnb
        out_copy(b).start()
    out_copy(B - 2).wait()
    out_copy(B - 1).wait()


@jax.jit
def _fused(grid, row_emb, col_emb, tok_emb, ln_w, ln_b):
    B, F, H, W, C = grid.shape
    out = pl.pallas_call(
        _body,
        in_specs=[
            pl.BlockSpec(memory_space=pl.ANY),
            pl.BlockSpec(memory_space=pltpu.VMEM),
            pl.BlockSpec(memory_space=pltpu.VMEM),
            pl.BlockSpec(memory_space=pltpu.VMEM),
            pl.BlockSpec(memory_space=pltpu.VMEM),
            pl.BlockSpec(memory_space=pltpu.VMEM),
        ],
        out_specs=pl.BlockSpec(memory_space=pl.ANY),
        out_shape=jax.ShapeDtypeStruct((B, H, W, C), grid.dtype),
        scratch_shapes=[
            pltpu.VMEM((F, H, W, C), grid.dtype),
            pltpu.VMEM((F, H, W, C), grid.dtype),
            pltpu.VMEM((F, H, W, C), grid.dtype),
            pltpu.VMEM((H, W, C), grid.dtype),
            pltpu.VMEM((H, W, C), grid.dtype),
            pltpu.SemaphoreType.DMA((_NBUF,)),
            pltpu.SemaphoreType.DMA((2,)),
        ],
    )(grid, row_emb[:H], col_emb[:W], tok_emb.reshape(1, C),
      ln_w.reshape(1, C), ln_b.reshape(1, C))
    return out.reshape(B, H * W, C)


def kernel(grid, row_emb, col_emb, tok_emb, ln_w, ln_b):
    return _fused(grid, row_emb, col_emb, tok_emb, ln_w, ln_b)
